# baseline (speedup 1.0000x reference)
import jax
import jax.numpy as jnp
from jax import lax
from jax.experimental import pallas as pl
from jax.experimental.pallas import tpu as pltpu
from jax.experimental.pallas import tpu_sc as plsc

PADDING_IDX = 0

NC = 2
NS = 16
NW = NC * NS
LANES = 16

SUB = 128
SUBS = 4
W = SUB * SUBS
NBUF = 3

BATCH = 16384
HIST = 200
EMBED_DIM = 32
DBLK = EMBED_DIM // 8
HD = HIST * DBLK


def _embed_body(idx_hbm, table_hbm, out_hbm, idx_v, rows_v, plane_v, *sems):
    isems = sems[0:NBUF]
    gsems = sems[NBUF:2 * NBUF]
    osems = sems[2 * NBUF:3 * NBUF]

    wid = lax.axis_index("s") * NC + lax.axis_index("c")
    bt0 = wid * SUBS

    def fire_i(h, s):
        pltpu.async_copy(
            idx_hbm.at[pl.ds((h // 8) * (BATCH // SUB) + bt0, SUBS), h % 8],
            idx_v.at[s], isems[s])

    def wait_i(s):
        pltpu.make_async_copy(idx_hbm.at[pl.ds(0, SUBS), 0],
                              idx_v.at[s], isems[s]).wait()

    def fire_g(s):
        for j in range(SUBS):
            pltpu.async_copy(table_hbm.at[idx_v.at[s, j]],
                             rows_v.at[s, pl.ds(j * SUB, SUB)], gsems[s])

    def wait_g(s):
        pltpu.make_async_copy(table_hbm.at[pl.ds(0, W)],
                              rows_v.at[s], gsems[s]).wait()

    def fire_o(h, s):
        for dblk in range(DBLK):
            pltpu.async_copy(
                plane_v.at[s, pl.ds(dblk * SUBS * 8, SUBS * 8), pl.ds(0, SUB)],
                out_hbm.at[h * DBLK + dblk, pl.ds(bt0 * 8, SUBS * 8)],
                osems[s])

    def wait_o(s):
        for dblk in range(DBLK):
            pltpu.make_async_copy(
                plane_v.at[s, pl.ds(dblk * SUBS * 8, SUBS * 8), pl.ds(0, SUB)],
                out_hbm.at[0, pl.ds(0, SUBS * 8)],
                osems[s]).wait()

    iota16 = lax.iota(jnp.int32, LANES)
    rp_vecs = [[((((c0 + iota16) >> 3) * SUBS + bl) * 8 + ((c0 + iota16) & 7))
                for bl in range(SUBS)] for c0 in (0, LANES)]

    def process(s):
        for bl in range(SUBS):
            def row_body(b, _, bl=bl):
                r = bl * SUB + b
                bvec = jnp.zeros((LANES,), jnp.int32) + b
                for ci in range(2):
                    vals = rows_v[s, r, pl.ds(ci * LANES, LANES)]
                    plsc.store_scatter(plane_v.at[s],
                                       [rp_vecs[ci][bl], bvec], vals)
                return ()

            lax.fori_loop(0, SUB, row_body, (), unroll=16)

        acc = jnp.full((LANES,), jnp.iinfo(jnp.int32).max, jnp.int32)
        for j in range(SUBS):
            for k in range(SUB // LANES):
                acc = jnp.minimum(acc, idx_v[s, j, pl.ds(k * LANES, LANES)])
        has_pad = jnp.min(acc) == PADDING_IDX

        @pl.when(has_pad)
        def _mask_pass():
            zeros = jnp.zeros((LANES,), jnp.float32)

            def group_body(g, _):
                j = g // (SUB // LANES)
                k = g % (SUB // LANES)
                idx16 = idx_v[s, j, pl.ds(k * LANES, LANES)]
                m = idx16 == PADDING_IDX
                b16 = k * LANES + lax.iota(jnp.int32, LANES)
                for d in range(EMBED_DIM):
                    rp = ((d // 8) * SUBS + j) * 8 + d % 8
                    rpv = jnp.zeros((LANES,), jnp.int32) + rp
                    plsc.store_scatter(plane_v.at[s], [rpv, b16],
                                       zeros, mask=m)
                return ()

            lax.fori_loop(0, W // LANES, group_body, (), unroll=False)

    def tile_iter(h, s):
        @pl.when(h < HIST)
        def _():
            @pl.when(h + 2 < HIST)
            def _():
                fire_i(h + 2, (s + 2) % NBUF)

            @pl.when(h + 1 < HIST)
            def _():
                wait_i((s + 1) % NBUF)
                fire_g((s + 1) % NBUF)

            wait_g(s)

            @pl.when(h >= NBUF)
            def _():
                wait_o(s)

            process(s)
            fire_o(h, s)

    fire_i(0, 0)
    fire_i(1, 1)
    wait_i(0)
    fire_g(0)

    def outer_body(t, _):
        for u in range(NBUF):
            tile_iter(t * NBUF + u, u)
        return ()

    lax.fori_loop(0, (HIST + NBUF - 1) // NBUF, outer_body, (), unroll=False)

    for s in range(NBUF):
        wait_o(s)


@jax.jit
def _embed(idx3d, table_rm):
    mesh = plsc.VectorSubcoreMesh(core_axis_name="c", subcore_axis_name="s")
    f = pl.kernel(
        _embed_body,
        out_type=jax.ShapeDtypeStruct((HD, (BATCH // SUB) * 8, SUB), jnp.float32),
        mesh=mesh,
        scratch_types=(
            [pltpu.VMEM((NBUF, SUBS, SUB), jnp.int32),
             pltpu.VMEM((NBUF, W, EMBED_DIM), jnp.float32),
             pltpu.VMEM((NBUF, DBLK * SUBS * 8, SUB + 1), jnp.float32)]
            + [pltpu.SemaphoreType.DMA] * (3 * NBUF)
        ),
        compiler_params=pltpu.CompilerParams(needs_layout_passes=False,
                                             use_tc_tiling_on_sc=False),
    )
    return f(idx3d, table_rm)


def kernel(token_ids, table):
    B, H = token_ids.shape
    D = table.shape[1]
    tids_t = token_ids.T.astype(jnp.int32)
    idx4 = tids_t.reshape(H // 8, 8, B // SUB, SUB)
    idx3d = idx4.transpose(0, 2, 1, 3).reshape(-1, 8, SUB)
    table_rm = table.astype(jnp.float32)
    out4 = _embed(idx3d, table_rm)
    out6 = out4.reshape(H, D // 8, B // SUB, 8, SUB)
    out = out6.transpose(2, 4, 0, 1, 3).reshape(B, H, D)
    return out


# speedup vs baseline: 1.0527x; 1.0527x over previous
import jax
import jax.numpy as jnp
from jax import lax
from jax.experimental import pallas as pl
from jax.experimental.pallas import tpu as pltpu
from jax.experimental.pallas import tpu_sc as plsc

PADDING_IDX = 0

NC = 2
NS = 16
NW = NC * NS
LANES = 16

SUB = 128
SUBS = 4
W = SUB * SUBS
NBUF = 3

BATCH = 16384
HIST = 200
EMBED_DIM = 32
DBLK = EMBED_DIM // 8
HD = HIST * DBLK


def _embed_body(idx_hbm, table_hbm, out_hbm, idx_v, rows_v, plane_v, *sems):
    isems = sems[0:NBUF]
    gsems = sems[NBUF:2 * NBUF]
    osems = sems[2 * NBUF:3 * NBUF]

    wid = lax.axis_index("s") * NC + lax.axis_index("c")
    bt0 = wid * SUBS

    def fire_i(h, s):
        pltpu.async_copy(
            idx_hbm.at[pl.ds((h // 8) * (BATCH // SUB) + bt0, SUBS), h % 8],
            idx_v.at[s], isems[s])

    def wait_i(s):
        pltpu.make_async_copy(idx_hbm.at[pl.ds(0, SUBS), 0],
                              idx_v.at[s], isems[s]).wait()

    def fire_g(s):
        for j in range(SUBS):
            pltpu.async_copy(table_hbm.at[idx_v.at[s, j]],
                             rows_v.at[s, pl.ds(j * SUB, SUB)], gsems[s])

    def wait_g(s):
        pltpu.make_async_copy(table_hbm.at[pl.ds(0, W)],
                              rows_v.at[s], gsems[s]).wait()

    def fire_o(h, s):
        for dblk in range(DBLK):
            pltpu.async_copy(
                plane_v.at[s, pl.ds(dblk * SUBS * 8, SUBS * 8), pl.ds(0, SUB)],
                out_hbm.at[h * DBLK + dblk, pl.ds(bt0 * 8, SUBS * 8)],
                osems[s])

    def wait_o(s):
        for dblk in range(DBLK):
            pltpu.make_async_copy(
                plane_v.at[s, pl.ds(dblk * SUBS * 8, SUBS * 8), pl.ds(0, SUB)],
                out_hbm.at[0, pl.ds(0, SUBS * 8)],
                osems[s]).wait()

    iota16 = lax.iota(jnp.int32, LANES)
    rp_vecs = [[((((c0 + iota16) >> 3) * SUBS + bl) * 8 + ((c0 + iota16) & 7))
                for bl in range(SUBS)] for c0 in (0, LANES)]

    def process(s):
        for bl in range(SUBS):
            def row_body(b, _, bl=bl):
                r = bl * SUB + b
                bvec = jnp.zeros((LANES,), jnp.int32) + b
                for ci in range(2):
                    vals = rows_v[s, r, pl.ds(ci * LANES, LANES)]
                    plsc.store_scatter(plane_v.at[s],
                                       [rp_vecs[ci][bl], bvec], vals)
                return ()

            lax.fori_loop(0, SUB, row_body, (), unroll=4)

        acc = jnp.full((LANES,), jnp.iinfo(jnp.int32).max, jnp.int32)
        for j in range(SUBS):
            for k in range(SUB // LANES):
                acc = jnp.minimum(acc, idx_v[s, j, pl.ds(k * LANES, LANES)])
        has_pad = jnp.min(acc) == PADDING_IDX

        @pl.when(has_pad)
        def _mask_pass():
            zeros = jnp.zeros((LANES,), jnp.float32)

            def group_body(g, _):
                j = g // (SUB // LANES)
                k = g % (SUB // LANES)
                idx16 = idx_v[s, j, pl.ds(k * LANES, LANES)]
                m = idx16 == PADDING_IDX
                b16 = k * LANES + lax.iota(jnp.int32, LANES)
                for d in range(EMBED_DIM):
                    rp = ((d // 8) * SUBS + j) * 8 + d % 8
                    rpv = jnp.zeros((LANES,), jnp.int32) + rp
                    plsc.store_scatter(plane_v.at[s], [rpv, b16],
                                       zeros, mask=m)
                return ()

            lax.fori_loop(0, W // LANES, group_body, (), unroll=False)

    def tile_iter(h, s):
        @pl.when(h < HIST)
        def _():
            @pl.when(h + 2 < HIST)
            def _():
                fire_i(h + 2, (s + 2) % NBUF)

            @pl.when(h + 1 < HIST)
            def _():
                wait_i((s + 1) % NBUF)
                fire_g((s + 1) % NBUF)

            wait_g(s)

            @pl.when(h >= NBUF)
            def _():
                wait_o(s)

            process(s)
            fire_o(h, s)

    fire_i(0, 0)
    fire_i(1, 1)
    wait_i(0)
    fire_g(0)

    def outer_body(t, _):
        for u in range(NBUF):
            tile_iter(t * NBUF + u, u)
        return ()

    lax.fori_loop(0, (HIST + NBUF - 1) // NBUF, outer_body, (), unroll=False)

    for s in range(NBUF):
        wait_o(s)


@jax.jit
def _embed(idx3d, table_rm):
    mesh = plsc.VectorSubcoreMesh(core_axis_name="c", subcore_axis_name="s")
    f = pl.kernel(
        _embed_body,
        out_type=jax.ShapeDtypeStruct((HD, (BATCH // SUB) * 8, SUB), jnp.float32),
        mesh=mesh,
        scratch_types=(
            [pltpu.VMEM((NBUF, SUBS, SUB), jnp.int32),
             pltpu.VMEM((NBUF, W, EMBED_DIM), jnp.float32),
             pltpu.VMEM((NBUF, DBLK * SUBS * 8, SUB + 1), jnp.float32)]
            + [pltpu.SemaphoreType.DMA] * (3 * NBUF)
        ),
        compiler_params=pltpu.CompilerParams(needs_layout_passes=False,
                                             use_tc_tiling_on_sc=False),
    )
    return f(idx3d, table_rm)


def kernel(token_ids, table):
    B, H = token_ids.shape
    D = table.shape[1]
    tids_t = token_ids.T.astype(jnp.int32)
    idx4 = tids_t.reshape(H // 8, 8, B // SUB, SUB)
    idx3d = idx4.transpose(0, 2, 1, 3).reshape(-1, 8, SUB)
    table_rm = table.astype(jnp.float32)
    out4 = _embed(idx3d, table_rm)
    out6 = out4.reshape(H, D // 8, B // SUB, 8, SUB)
    out = out6.transpose(2, 4, 0, 1, 3).reshape(B, H, D)
    return out


# speedup vs baseline: 1.6162x; 1.5353x over previous
import jax
import jax.numpy as jnp
from jax import lax
from jax.experimental import pallas as pl
from jax.experimental.pallas import tpu as pltpu
from jax.experimental.pallas import tpu_sc as plsc

PADDING_IDX = 0

NC = 2
NS = 16
NW = NC * NS
LANES = 16

SUB = 128
SUBS = 4
W = SUB * SUBS
NBUF = 3

BATCH = 16384
HIST = 200
EMBED_DIM = 32
DBLK = EMBED_DIM // 8
HD = HIST * DBLK


def _embed_body(idx_hbm, table_hbm, out_hbm, idx_v, rows_v, plane_v, *sems):
    isems = sems[0:NBUF]
    gsems = sems[NBUF:2 * NBUF]
    osems = sems[2 * NBUF:3 * NBUF]

    wid = lax.axis_index("s") * NC + lax.axis_index("c")
    bt0 = wid * SUBS

    def fire_i(h, s):
        pltpu.async_copy(
            idx_hbm.at[pl.ds((h // 8) * (BATCH // SUB) + bt0, SUBS), h % 8],
            idx_v.at[s], isems[s])

    def wait_i(s):
        pltpu.make_async_copy(idx_hbm.at[pl.ds(0, SUBS), 0],
                              idx_v.at[s], isems[s]).wait()

    def fire_g(s):
        for j in range(SUBS):
            pltpu.async_copy(table_hbm.at[idx_v.at[s, j]],
                             rows_v.at[s, pl.ds(j * SUB, SUB)], gsems[s])

    def wait_g(s):
        pltpu.make_async_copy(table_hbm.at[pl.ds(0, W)],
                              rows_v.at[s], gsems[s]).wait()

    def fire_o(h, s):
        for dblk in range(DBLK):
            pltpu.async_copy(
                plane_v.at[s, pl.ds(dblk * SUBS * 8, SUBS * 8), pl.ds(0, SUB)],
                out_hbm.at[h * DBLK + dblk, pl.ds(bt0 * 8, SUBS * 8)],
                osems[s])

    def wait_o(s):
        for dblk in range(DBLK):
            pltpu.make_async_copy(
                plane_v.at[s, pl.ds(dblk * SUBS * 8, SUBS * 8), pl.ds(0, SUB)],
                out_hbm.at[0, pl.ds(0, SUBS * 8)],
                osems[s]).wait()

    iota16 = lax.iota(jnp.int32, LANES)
    rp_vecs = [[((((c0 + iota16) >> 3) * SUBS + bl) * 8 + ((c0 + iota16) & 7))
                for bl in range(SUBS)] for c0 in (0, LANES)]

    def process(s):
        for bl in range(SUBS):
            def quad_body(q, _, bl=bl):
                b0 = q * 4
                base = jnp.zeros((LANES,), jnp.int32) + b0
                bvecs = [base + i for i in range(4)]
                vals = [rows_v[s, bl * SUB + b0 + i, pl.ds(ci * LANES, LANES)]
                        for i in range(4) for ci in range(2)]
                for i in range(4):
                    for ci in range(2):
                        plsc.store_scatter(plane_v.at[s],
                                           [rp_vecs[ci][bl], bvecs[i]],
                                           vals[i * 2 + ci])
                return ()

            lax.fori_loop(0, SUB // 4, quad_body, (), unroll=2)

        acc = jnp.full((LANES,), jnp.iinfo(jnp.int32).max, jnp.int32)
        for j in range(SUBS):
            for k in range(SUB // LANES):
                acc = jnp.minimum(acc, idx_v[s, j, pl.ds(k * LANES, LANES)])
        has_pad = jnp.min(acc) == PADDING_IDX

        @pl.when(has_pad)
        def _mask_pass():
            zeros = jnp.zeros((LANES,), jnp.float32)

            def group_body(g, _):
                j = g // (SUB // LANES)
                k = g % (SUB // LANES)
                idx16 = idx_v[s, j, pl.ds(k * LANES, LANES)]
                m = idx16 == PADDING_IDX
                b16 = k * LANES + lax.iota(jnp.int32, LANES)
                for d in range(EMBED_DIM):
                    rp = ((d // 8) * SUBS + j) * 8 + d % 8
                    rpv = jnp.zeros((LANES,), jnp.int32) + rp
                    plsc.store_scatter(plane_v.at[s], [rpv, b16],
                                       zeros, mask=m)
                return ()

            lax.fori_loop(0, W // LANES, group_body, (), unroll=False)

    def tile_iter(h, s):
        @pl.when(h < HIST)
        def _():
            @pl.when(h + 2 < HIST)
            def _():
                fire_i(h + 2, (s + 2) % NBUF)

            @pl.when(h + 1 < HIST)
            def _():
                wait_i((s + 1) % NBUF)
                fire_g((s + 1) % NBUF)

            wait_g(s)

            @pl.when(h >= NBUF)
            def _():
                wait_o(s)

            process(s)
            fire_o(h, s)

    fire_i(0, 0)
    fire_i(1, 1)
    wait_i(0)
    fire_g(0)

    def outer_body(t, _):
        for u in range(NBUF):
            tile_iter(t * NBUF + u, u)
        return ()

    lax.fori_loop(0, (HIST + NBUF - 1) // NBUF, outer_body, (), unroll=False)

    for s in range(NBUF):
        wait_o(s)


@jax.jit
def _embed(idx3d, table_rm):
    mesh = plsc.VectorSubcoreMesh(core_axis_name="c", subcore_axis_name="s")
    f = pl.kernel(
        _embed_body,
        out_type=jax.ShapeDtypeStruct((HD, (BATCH // SUB) * 8, SUB), jnp.float32),
        mesh=mesh,
        scratch_types=(
            [pltpu.VMEM((NBUF, SUBS, SUB), jnp.int32),
             pltpu.VMEM((NBUF, W, EMBED_DIM), jnp.float32),
             pltpu.VMEM((NBUF, DBLK * SUBS * 8, SUB + 1), jnp.float32)]
            + [pltpu.SemaphoreType.DMA] * (3 * NBUF)
        ),
        compiler_params=pltpu.CompilerParams(needs_layout_passes=False,
                                             use_tc_tiling_on_sc=False),
    )
    return f(idx3d, table_rm)


def kernel(token_ids, table):
    B, H = token_ids.shape
    D = table.shape[1]
    tids_t = token_ids.T.astype(jnp.int32)
    idx4 = tids_t.reshape(H // 8, 8, B // SUB, SUB)
    idx3d = idx4.transpose(0, 2, 1, 3).reshape(-1, 8, SUB)
    table_rm = table.astype(jnp.float32)
    out4 = _embed(idx3d, table_rm)
    out6 = out4.reshape(H, D // 8, B // SUB, 8, SUB)
    out = out6.transpose(2, 4, 0, 1, 3).reshape(B, H, D)
    return out


# speedup vs baseline: 1.6208x; 1.0028x over previous
import jax
import jax.numpy as jnp
from jax import lax
from jax.experimental import pallas as pl
from jax.experimental.pallas import tpu as pltpu
from jax.experimental.pallas import tpu_sc as plsc

PADDING_IDX = 0

NC = 2
NS = 16
NW = NC * NS
LANES = 16

SUB = 128
SUBS = 4
W = SUB * SUBS
NBUF = 3

BATCH = 16384
HIST = 200
EMBED_DIM = 32
DBLK = EMBED_DIM // 8
HD = HIST * DBLK


def _embed_body(idx_hbm, table_hbm, out_hbm, idx_v, rows_v, plane_v, *sems):
    isems = sems[0:NBUF]
    gsems = sems[NBUF:2 * NBUF]
    osems = sems[2 * NBUF:3 * NBUF]

    wid = lax.axis_index("s") * NC + lax.axis_index("c")
    bt0 = wid * SUBS

    def fire_i(h, s):
        pltpu.async_copy(
            idx_hbm.at[pl.ds((h // 8) * (BATCH // SUB) + bt0, SUBS), h % 8],
            idx_v.at[s], isems[s])

    def wait_i(s):
        pltpu.make_async_copy(idx_hbm.at[pl.ds(0, SUBS), 0],
                              idx_v.at[s], isems[s]).wait()

    def fire_g(s):
        for j in range(SUBS):
            pltpu.async_copy(table_hbm.at[idx_v.at[s, j]],
                             rows_v.at[s, pl.ds(j * SUB, SUB)], gsems[s])

    def wait_g(s):
        pltpu.make_async_copy(table_hbm.at[pl.ds(0, W)],
                              rows_v.at[s], gsems[s]).wait()

    def fire_o(h, s):
        for dblk in range(DBLK):
            pltpu.async_copy(
                plane_v.at[s, pl.ds(dblk * SUBS * 8, SUBS * 8), pl.ds(0, SUB)],
                out_hbm.at[h * DBLK + dblk, pl.ds(bt0 * 8, SUBS * 8)],
                osems[s])

    def wait_o(s):
        for dblk in range(DBLK):
            pltpu.make_async_copy(
                plane_v.at[s, pl.ds(dblk * SUBS * 8, SUBS * 8), pl.ds(0, SUB)],
                out_hbm.at[0, pl.ds(0, SUBS * 8)],
                osems[s]).wait()

    iota16 = lax.iota(jnp.int32, LANES)
    rp_vecs = [[((((c0 + iota16) >> 3) * SUBS + bl) * 8 + ((c0 + iota16) & 7))
                for bl in range(SUBS)] for c0 in (0, LANES)]

    def process(s):
        for bl in range(SUBS):
            def quad_body(q, _, bl=bl):
                b0 = q * 8
                base = jnp.zeros((LANES,), jnp.int32) + b0
                bvecs = [base + i for i in range(8)]
                vals = [rows_v[s, bl * SUB + b0 + i, pl.ds(ci * LANES, LANES)]
                        for i in range(8) for ci in range(2)]
                for i in range(8):
                    for ci in range(2):
                        plsc.store_scatter(plane_v.at[s],
                                           [rp_vecs[ci][bl], bvecs[i]],
                                           vals[i * 2 + ci])
                return ()

            lax.fori_loop(0, SUB // 8, quad_body, (), unroll=1)

        acc = jnp.full((LANES,), jnp.iinfo(jnp.int32).max, jnp.int32)
        for j in range(SUBS):
            for k in range(SUB // LANES):
                acc = jnp.minimum(acc, idx_v[s, j, pl.ds(k * LANES, LANES)])
        has_pad = jnp.min(acc) == PADDING_IDX

        @pl.when(has_pad)
        def _mask_pass():
            zeros = jnp.zeros((LANES,), jnp.float32)

            def group_body(g, _):
                j = g // (SUB // LANES)
                k = g % (SUB // LANES)
                idx16 = idx_v[s, j, pl.ds(k * LANES, LANES)]
                m = idx16 == PADDING_IDX
                b16 = k * LANES + lax.iota(jnp.int32, LANES)
                for d in range(EMBED_DIM):
                    rp = ((d // 8) * SUBS + j) * 8 + d % 8
                    rpv = jnp.zeros((LANES,), jnp.int32) + rp
                    plsc.store_scatter(plane_v.at[s], [rpv, b16],
                                       zeros, mask=m)
                return ()

            lax.fori_loop(0, W // LANES, group_body, (), unroll=False)

    def tile_iter(h, s):
        @pl.when(h < HIST)
        def _():
            @pl.when(h + 2 < HIST)
            def _():
                fire_i(h + 2, (s + 2) % NBUF)

            @pl.when(h + 1 < HIST)
            def _():
                wait_i((s + 1) % NBUF)
                fire_g((s + 1) % NBUF)

            wait_g(s)

            @pl.when(h >= NBUF)
            def _():
                wait_o(s)

            process(s)
            fire_o(h, s)

    fire_i(0, 0)
    fire_i(1, 1)
    wait_i(0)
    fire_g(0)

    def outer_body(t, _):
        for u in range(NBUF):
            tile_iter(t * NBUF + u, u)
        return ()

    lax.fori_loop(0, (HIST + NBUF - 1) // NBUF, outer_body, (), unroll=False)

    for s in range(NBUF):
        wait_o(s)


@jax.jit
def _embed(idx3d, table_rm):
    mesh = plsc.VectorSubcoreMesh(core_axis_name="c", subcore_axis_name="s")
    f = pl.kernel(
        _embed_body,
        out_type=jax.ShapeDtypeStruct((HD, (BATCH // SUB) * 8, SUB), jnp.float32),
        mesh=mesh,
        scratch_types=(
            [pltpu.VMEM((NBUF, SUBS, SUB), jnp.int32),
             pltpu.VMEM((NBUF, W, EMBED_DIM), jnp.float32),
             pltpu.VMEM((NBUF, DBLK * SUBS * 8, SUB + 1), jnp.float32)]
            + [pltpu.SemaphoreType.DMA] * (3 * NBUF)
        ),
        compiler_params=pltpu.CompilerParams(needs_layout_passes=False,
                                             use_tc_tiling_on_sc=False),
    )
    return f(idx3d, table_rm)


def kernel(token_ids, table):
    B, H = token_ids.shape
    D = table.shape[1]
    tids_t = token_ids.T.astype(jnp.int32)
    idx4 = tids_t.reshape(H // 8, 8, B // SUB, SUB)
    idx3d = idx4.transpose(0, 2, 1, 3).reshape(-1, 8, SUB)
    table_rm = table.astype(jnp.float32)
    out4 = _embed(idx3d, table_rm)
    out6 = out4.reshape(H, D // 8, B // SUB, 8, SUB)
    out = out6.transpose(2, 4, 0, 1, 3).reshape(B, H, D)
    return out
